# baseline (speedup 1.0000x reference)
import jax
import jax.numpy as jnp
from jax import lax
from jax.experimental import pallas as pl
from jax.experimental.pallas import tpu as pltpu
from jax.experimental.pallas import tpu_sc as plsc

_N = 10000
_E = 320000
_D_IN = 128
_D_HID = 32
_D_OUT = 16

_NC = 2
_NS = 16
_NW = _NC * _NS
_CHUNK = 128
_NB = 8
_CPW = 80
_EW = _CPW * _CHUNK
_E_PAD = _NW * _EW
_N_PAD = 10240
_STRIPE = _N_PAD // _NS


def _deg_body(dst_hbm, deg_hbm, dst_v, ones_v, buf_v, deg_sh, ssem):
    c = lax.axis_index("c")
    s = lax.axis_index("s")
    wid = c * _NS + s

    def fill_ones(i, _):
        ones_v[pl.ds(i * 16, 16)] = jnp.full((16,), 1.0, jnp.float32)
        return 0

    lax.fori_loop(0, _CHUNK // 16, fill_ones, 0)

    def fill_zero(i, _):
        buf_v[pl.ds(i * 16, 16)] = jnp.zeros((16,), jnp.float32)
        return 0

    lax.fori_loop(0, _STRIPE // 16, fill_zero, 0)

    stripe = pl.ds(s * _STRIPE, _STRIPE)
    pltpu.sync_copy(buf_v, deg_sh.at[stripe])
    pltpu.sync_copy(dst_hbm.at[wid], dst_v)
    plsc.subcore_barrier()

    def body(g, _):
        sds = []
        for b in range(_NB):
            j = g * _NB + b
            sds.append(
                pltpu.async_copy(ones_v, deg_sh.at[dst_v.at[j]], ssem, add=True)
            )
        for d_ in sds:
            d_.wait()
        return 0

    lax.fori_loop(0, _CPW // _NB, body, 0)
    plsc.subcore_barrier()
    pltpu.sync_copy(deg_sh.at[stripe], buf_v)
    pltpu.sync_copy(buf_v, deg_hbm.at[c, stripe])


def _make_deg():
    mesh = plsc.VectorSubcoreMesh(core_axis_name="c", subcore_axis_name="s")
    return pl.kernel(
        _deg_body,
        out_type=jax.ShapeDtypeStruct((_NC, _N_PAD), jnp.float32),
        mesh=mesh,
        scratch_types=[
            pltpu.VMEM((_CPW, _CHUNK), jnp.int32),
            pltpu.VMEM((_CHUNK,), jnp.float32),
            pltpu.VMEM((_STRIPE,), jnp.float32),
            pltpu.VMEM_SHARED((_N_PAD,), jnp.float32),
            pltpu.SemaphoreType.DMA,
        ],
    )


def _msg_body(h_hbm, src_hbm, dst_hbm, out_hbm, src_v, dst_v, rows_v, stage_v,
              acc_sh, gsem, ssem):
    c = lax.axis_index("c")
    s = lax.axis_index("s")
    wid = c * _NS + s
    stripe = pl.ds(s * _STRIPE, _STRIPE)

    pltpu.sync_copy(h_hbm.at[stripe], stage_v)
    pltpu.sync_copy(stage_v, acc_sh.at[stripe])
    pltpu.sync_copy(src_hbm.at[wid], src_v)
    pltpu.sync_copy(dst_hbm.at[wid], dst_v)
    plsc.subcore_barrier()

    def body(g, _):
        gds = []
        for b in range(_NB):
            j = g * _NB + b
            gds.append(
                pltpu.async_copy(h_hbm.at[src_v.at[j]], rows_v.at[b], gsem)
            )
        for d_ in gds:
            d_.wait()
        sds = []
        for b in range(_NB):
            j = g * _NB + b
            sds.append(
                pltpu.async_copy(
                    rows_v.at[b], acc_sh.at[dst_v.at[j]], ssem, add=True
                )
            )
        for d_ in sds:
            d_.wait()
        return 0

    lax.fori_loop(0, _CPW // _NB, body, 0)
    plsc.subcore_barrier()
    pltpu.sync_copy(acc_sh.at[stripe], stage_v)
    pltpu.sync_copy(stage_v, out_hbm.at[c, stripe])


def _make_msg(d):
    mesh = plsc.VectorSubcoreMesh(core_axis_name="c", subcore_axis_name="s")
    return pl.kernel(
        _msg_body,
        out_type=jax.ShapeDtypeStruct((_NC, _N_PAD, d), jnp.float32),
        mesh=mesh,
        compiler_params=pltpu.CompilerParams(use_tc_tiling_on_sc=False),
        scratch_types=[
            pltpu.VMEM((_CPW, _CHUNK), jnp.int32),
            pltpu.VMEM((_CPW, _CHUNK), jnp.int32),
            pltpu.VMEM((_NB, _CHUNK, d), jnp.float32),
            pltpu.VMEM((_STRIPE, d), jnp.float32),
            pltpu.VMEM_SHARED((_N_PAD, d), jnp.float32),
            pltpu.SemaphoreType.DMA,
            pltpu.SemaphoreType.DMA,
        ],
    )


def _tc_a_body(degt_ref, x_ref, w1_ref, dis_ref, h_ref):
    deg = degt_ref[:, 0:1] + degt_ref[:, 1:2] + 1.0
    dis = lax.rsqrt(deg)
    h = jnp.dot(x_ref[...], w1_ref[...], preferred_element_type=jnp.float32)
    dis_ref[...] = dis
    h_ref[...] = h * dis


def _tc_b_body(p_ref, h_ref, dis_ref, b1_ref, w2_ref, out_ref):
    dis = dis_ref[...]
    acc = p_ref[0, :, :] + p_ref[1, :, :] - h_ref[...]
    h1 = jnp.maximum(dis * acc + b1_ref[...], 0.0)
    out_ref[...] = (
        jnp.dot(h1, w2_ref[...], preferred_element_type=jnp.float32) * dis
    )


def _tc_c_body(p_ref, h_ref, dis_ref, b2_ref, out_ref):
    out_ref[...] = (
        dis_ref[...] * (p_ref[0, :, :] + p_ref[1, :, :] - h_ref[...])
        + b2_ref[...]
    )


_tc_a = pl.pallas_call(
    _tc_a_body,
    out_shape=[
        jax.ShapeDtypeStruct((_N_PAD, 1), jnp.float32),
        jax.ShapeDtypeStruct((_N_PAD, _D_HID), jnp.float32),
    ],
)

_tc_b = pl.pallas_call(
    _tc_b_body,
    out_shape=jax.ShapeDtypeStruct((_N_PAD, _D_OUT), jnp.float32),
)

_tc_c = pl.pallas_call(
    _tc_c_body,
    out_shape=jax.ShapeDtypeStruct((_N_PAD, _D_OUT), jnp.float32),
)


def kernel(x, edge_index, W1, b1, W2, b2):
    src = edge_index[0]
    dst = edge_index[1]
    pad = jnp.full((_E_PAD - _E,), _N, jnp.int32)
    src3 = jnp.concatenate([src, pad]).reshape(_NW, _CPW, _CHUNK)
    dst3 = jnp.concatenate([dst, pad]).reshape(_NW, _CPW, _CHUNK)
    x_pad = jnp.zeros((_N_PAD, _D_IN), jnp.float32).at[:_N].set(x)

    degp = _make_deg()(dst3)
    degt = degp.T

    dis, h1s = _tc_a(degt, x_pad, W1)
    p1 = _make_msg(_D_HID)(h1s, src3, dst3)
    h2s = _tc_b(p1, h1s, dis, b1.reshape(1, _D_HID), W2)
    p2 = _make_msg(_D_OUT)(h2s, src3, dst3)
    out = _tc_c(p2, h2s, dis, b2.reshape(1, _D_OUT))
    return out[:_N]


# speedup vs baseline: 1.9214x; 1.9214x over previous
import jax
import jax.numpy as jnp
from jax import lax
from jax.experimental import pallas as pl
from jax.experimental.pallas import tpu as pltpu
from jax.experimental.pallas import tpu_sc as plsc

_N = 10000
_E = 320000
_D_IN = 128
_D_HID = 32
_D_OUT = 16

_NC = 2
_NS = 16
_NW = _NC * _NS
_CHUNK = 128
_NB = 8
_CPW = 80
_EW = _CPW * _CHUNK
_E_PAD = _NW * _EW
_N_PAD = 10240
_STRIPE = _N_PAD // _NS


def _deg_body(dst_hbm, deg_hbm, dst_v, ones_v, buf_v, deg_sh, ssem):
    c = lax.axis_index("c")
    s = lax.axis_index("s")
    wid = c * _NS + s

    def fill_ones(i, _):
        ones_v[pl.ds(i * 16, 16)] = jnp.full((16,), 1.0, jnp.float32)
        return 0

    lax.fori_loop(0, _CHUNK // 16, fill_ones, 0)

    def fill_zero(i, _):
        buf_v[pl.ds(i * 16, 16)] = jnp.zeros((16,), jnp.float32)
        return 0

    lax.fori_loop(0, _STRIPE // 16, fill_zero, 0)

    stripe = pl.ds(s * _STRIPE, _STRIPE)
    pltpu.sync_copy(buf_v, deg_sh.at[stripe])
    pltpu.sync_copy(dst_hbm.at[wid], dst_v)
    plsc.subcore_barrier()

    def body(g, _):
        sds = []
        for b in range(_NB):
            j = g * _NB + b
            sds.append(
                pltpu.async_copy(ones_v, deg_sh.at[dst_v.at[j]], ssem, add=True)
            )
        for d_ in sds:
            d_.wait()
        return 0

    lax.fori_loop(0, _CPW // _NB, body, 0)
    plsc.subcore_barrier()
    pltpu.sync_copy(deg_sh.at[stripe], buf_v)
    pltpu.sync_copy(buf_v, deg_hbm.at[c, stripe])


def _make_deg():
    mesh = plsc.VectorSubcoreMesh(core_axis_name="c", subcore_axis_name="s")
    return pl.kernel(
        _deg_body,
        out_type=jax.ShapeDtypeStruct((_NC, _N_PAD), jnp.float32),
        mesh=mesh,
        scratch_types=[
            pltpu.VMEM((_CPW, _CHUNK), jnp.int32),
            pltpu.VMEM((_CHUNK,), jnp.float32),
            pltpu.VMEM((_STRIPE,), jnp.float32),
            pltpu.VMEM_SHARED((_N_PAD,), jnp.float32),
            pltpu.SemaphoreType.DMA,
        ],
    )


def _msg_body(h_hbm, src_hbm, dst_hbm, out_hbm, src_v, dst_v, rows_v, stage_v,
              acc_sh, gsem, ssem):
    c = lax.axis_index("c")
    s = lax.axis_index("s")
    wid = c * _NS + s
    stripe = pl.ds(s * _STRIPE, _STRIPE)

    pltpu.sync_copy(h_hbm.at[stripe], stage_v)
    pltpu.sync_copy(stage_v, acc_sh.at[stripe])
    pltpu.sync_copy(src_hbm.at[wid], src_v)
    pltpu.sync_copy(dst_hbm.at[wid], dst_v)
    plsc.subcore_barrier()

    def body(g, _):
        gds = []
        for b in range(_NB):
            j = g * _NB + b
            gds.append(
                pltpu.async_copy(h_hbm.at[src_v.at[j]], rows_v.at[b], gsem)
            )
        for d_ in gds:
            d_.wait()
        sds = []
        for b in range(_NB):
            j = g * _NB + b
            sds.append(
                pltpu.async_copy(
                    rows_v.at[b], acc_sh.at[dst_v.at[j]], ssem, add=True
                )
            )
        for d_ in sds:
            d_.wait()
        return 0

    lax.fori_loop(0, _CPW // _NB, body, 0)
    plsc.subcore_barrier()
    pltpu.sync_copy(acc_sh.at[stripe], stage_v)
    pltpu.sync_copy(stage_v, out_hbm.at[c, stripe])


def _make_msg(d):
    mesh = plsc.VectorSubcoreMesh(core_axis_name="c", subcore_axis_name="s")
    return pl.kernel(
        _msg_body,
        out_type=jax.ShapeDtypeStruct((_NC, _N_PAD, d), jnp.float32),
        mesh=mesh,
        compiler_params=pltpu.CompilerParams(use_tc_tiling_on_sc=False),
        scratch_types=[
            pltpu.VMEM((_CPW, _CHUNK), jnp.int32),
            pltpu.VMEM((_CPW, _CHUNK), jnp.int32),
            pltpu.VMEM((_NB, _CHUNK, d), jnp.float32),
            pltpu.VMEM((_STRIPE, d), jnp.float32),
            pltpu.VMEM_SHARED((_N_PAD, d), jnp.float32),
            pltpu.SemaphoreType.DMA,
            pltpu.SemaphoreType.DMA,
        ],
    )


def _tc_a_body(degt_ref, x_ref, w1_ref, dis_ref, h_ref):
    deg = degt_ref[:, 0:1] + degt_ref[:, 1:2] + 1.0
    dis = lax.rsqrt(deg)
    h = jnp.dot(x_ref[...], w1_ref[...], preferred_element_type=jnp.float32)
    dis_ref[...] = dis
    h_ref[...] = h * dis


def _tc_b_body(p_ref, h_ref, dis_ref, b1_ref, w2_ref, out_ref):
    dis = dis_ref[...]
    acc = p_ref[0, :, :] + p_ref[1, :, :] - h_ref[...]
    h1 = jnp.maximum(dis * acc + b1_ref[...], 0.0)
    out_ref[...] = (
        jnp.dot(h1, w2_ref[...], preferred_element_type=jnp.float32) * dis
    )


def _tc_c_body(p_ref, h_ref, dis_ref, b2_ref, out_ref):
    out_ref[...] = (
        dis_ref[...] * (p_ref[0, :, :] + p_ref[1, :, :] - h_ref[...])
        + b2_ref[...]
    )


_tc_a = pl.pallas_call(
    _tc_a_body,
    out_shape=[
        jax.ShapeDtypeStruct((_N_PAD, 1), jnp.float32),
        jax.ShapeDtypeStruct((_N_PAD, _D_HID), jnp.float32),
    ],
)

_tc_b = pl.pallas_call(
    _tc_b_body,
    out_shape=jax.ShapeDtypeStruct((_N_PAD, _D_OUT), jnp.float32),
)

_tc_c = pl.pallas_call(
    _tc_c_body,
    out_shape=jax.ShapeDtypeStruct((_N_PAD, _D_OUT), jnp.float32),
)


def kernel(x, edge_index, W1, b1, W2, b2):
    src = edge_index[0]
    dst = edge_index[1]
    pad = _N + jnp.arange(_E_PAD - _E, dtype=jnp.int32) % (_N_PAD - _N)
    src3 = jnp.concatenate([src, pad]).reshape(_NW, _CPW, _CHUNK)
    dst3 = jnp.concatenate([dst, pad]).reshape(_NW, _CPW, _CHUNK)
    x_pad = jnp.zeros((_N_PAD, _D_IN), jnp.float32).at[:_N].set(x)

    degp = _make_deg()(dst3)
    degt = degp.T

    dis, h1s = _tc_a(degt, x_pad, W1)
    p1 = _make_msg(_D_HID)(h1s, src3, dst3)
    h2s = _tc_b(p1, h1s, dis, b1.reshape(1, _D_HID), W2)
    p2 = _make_msg(_D_OUT)(h2s, src3, dst3)
    out = _tc_c(p2, h2s, dis, b2.reshape(1, _D_OUT))
    return out[:_N]


# speedup vs baseline: 2.0736x; 1.0792x over previous
import jax
import jax.numpy as jnp
from jax import lax
from jax.experimental import pallas as pl
from jax.experimental.pallas import tpu as pltpu
from jax.experimental.pallas import tpu_sc as plsc

_N = 10000
_E = 320000
_D_IN = 128
_D_HID = 32
_D_OUT = 16

_NC = 2
_NS = 16
_NW = _NC * _NS
_CHUNK = 128
_NB = 6
_NCHUNKS = _E // _CHUNK
_CPW = _NCHUNKS // _NW
_XTRA = _NCHUNKS - _CPW * _NW
_N_PAD = 10240
_STRIPE = _N_PAD // _NS


def _wbase(wid):
    return wid * _CPW + jnp.minimum(wid, _XTRA)


def _load_idx(ei_hbm, which, wid, idx_v):
    base = _wbase(wid)
    pltpu.sync_copy(
        ei_hbm.at[which, pl.ds(base, _CPW)], idx_v.at[pl.ds(0, _CPW)]
    )

    @pl.when(wid < _XTRA)
    def _():
        pltpu.sync_copy(
            ei_hbm.at[which, pl.ds(base + _CPW, 1)], idx_v.at[pl.ds(_CPW, 1)]
        )


def _deg_body(ei_hbm, deg_hbm, dst_v, ones_v, buf_v, deg_sh, ssem):
    c = lax.axis_index("c")
    s = lax.axis_index("s")
    wid = c * _NS + s

    def fill_ones(i, _):
        ones_v[pl.ds(i * 16, 16)] = jnp.full((16,), 1.0, jnp.float32)
        return 0

    lax.fori_loop(0, _CHUNK // 16, fill_ones, 0)

    def fill_zero(i, _):
        buf_v[pl.ds(i * 16, 16)] = jnp.zeros((16,), jnp.float32)
        return 0

    lax.fori_loop(0, _STRIPE // 16, fill_zero, 0)

    stripe = pl.ds(s * _STRIPE, _STRIPE)
    pltpu.sync_copy(buf_v, deg_sh.at[stripe])
    _load_idx(ei_hbm, 1, wid, dst_v)
    plsc.subcore_barrier()

    def body(g, _):
        sds = []
        for b in range(_NB):
            j = g * _NB + b
            sds.append(
                pltpu.async_copy(ones_v, deg_sh.at[dst_v.at[j]], ssem, add=True)
            )
        for d_ in sds:
            d_.wait()
        return 0

    lax.fori_loop(0, _CPW // _NB, body, 0)

    @pl.when(wid < _XTRA)
    def _():
        pltpu.sync_copy(ones_v, deg_sh.at[dst_v.at[_CPW]], add=True)

    plsc.subcore_barrier()
    pltpu.sync_copy(deg_sh.at[stripe], buf_v)
    pltpu.sync_copy(buf_v, deg_hbm.at[c, stripe])


def _make_deg():
    mesh = plsc.VectorSubcoreMesh(core_axis_name="c", subcore_axis_name="s")
    return pl.kernel(
        _deg_body,
        out_type=jax.ShapeDtypeStruct((_NC, _N_PAD), jnp.float32),
        mesh=mesh,
        compiler_params=pltpu.CompilerParams(use_tc_tiling_on_sc=False),
        scratch_types=[
            pltpu.VMEM((_CPW + 1, _CHUNK), jnp.int32),
            pltpu.VMEM((_CHUNK,), jnp.float32),
            pltpu.VMEM((_STRIPE,), jnp.float32),
            pltpu.VMEM_SHARED((_N_PAD,), jnp.float32),
            pltpu.SemaphoreType.DMA,
        ],
    )


def _msg_body(h_hbm, ei_hbm, out_hbm, src_v, dst_v, rows_v, stage_v, acc_sh,
              gsem, ssem):
    c = lax.axis_index("c")
    s = lax.axis_index("s")
    wid = c * _NS + s
    stripe = pl.ds(s * _STRIPE, _STRIPE)

    pltpu.sync_copy(h_hbm.at[stripe], stage_v)
    pltpu.sync_copy(stage_v, acc_sh.at[stripe])
    _load_idx(ei_hbm, 0, wid, src_v)
    _load_idx(ei_hbm, 1, wid, dst_v)
    plsc.subcore_barrier()

    def body(g, _):
        gds = []
        for b in range(_NB):
            j = g * _NB + b
            gds.append(
                pltpu.async_copy(h_hbm.at[src_v.at[j]], rows_v.at[b], gsem)
            )
        for d_ in gds:
            d_.wait()
        sds = []
        for b in range(_NB):
            j = g * _NB + b
            sds.append(
                pltpu.async_copy(
                    rows_v.at[b], acc_sh.at[dst_v.at[j]], ssem, add=True
                )
            )
        for d_ in sds:
            d_.wait()
        return 0

    lax.fori_loop(0, _CPW // _NB, body, 0)

    @pl.when(wid < _XTRA)
    def _():
        pltpu.sync_copy(h_hbm.at[src_v.at[_CPW]], rows_v.at[0])
        pltpu.sync_copy(rows_v.at[0], acc_sh.at[dst_v.at[_CPW]], add=True)

    plsc.subcore_barrier()
    pltpu.sync_copy(acc_sh.at[stripe], stage_v)
    pltpu.sync_copy(stage_v, out_hbm.at[c, stripe])


def _make_msg(d):
    mesh = plsc.VectorSubcoreMesh(core_axis_name="c", subcore_axis_name="s")
    return pl.kernel(
        _msg_body,
        out_type=jax.ShapeDtypeStruct((_NC, _N_PAD, d), jnp.float32),
        mesh=mesh,
        compiler_params=pltpu.CompilerParams(use_tc_tiling_on_sc=False),
        scratch_types=[
            pltpu.VMEM((_CPW + 1, _CHUNK), jnp.int32),
            pltpu.VMEM((_CPW + 1, _CHUNK), jnp.int32),
            pltpu.VMEM((_NB, _CHUNK, d), jnp.float32),
            pltpu.VMEM((_STRIPE, d), jnp.float32),
            pltpu.VMEM_SHARED((_N_PAD, d), jnp.float32),
            pltpu.SemaphoreType.DMA,
            pltpu.SemaphoreType.DMA,
        ],
    )


def _tc_a_body(degp_ref, x_ref, w1_ref, dis_ref, h_ref):
    deg = degp_ref[0, :] + degp_ref[1, :] + 1.0
    dis = jnp.reshape(lax.rsqrt(deg), (_N_PAD, 1))
    h = jnp.dot(x_ref[...], w1_ref[...], preferred_element_type=jnp.float32)
    dis_ref[...] = dis
    h_ref[pl.ds(0, _N), :] = h * dis[:_N, :]
    h_ref[pl.ds(_N, _N_PAD - _N), :] = jnp.zeros(
        (_N_PAD - _N, _D_HID), jnp.float32
    )


def _tc_b_body(p_ref, h_ref, dis_ref, b1_ref, w2_ref, out_ref):
    dis = dis_ref[...]
    acc = p_ref[0, :, :] + p_ref[1, :, :] - h_ref[...]
    h1 = jnp.maximum(dis * acc + b1_ref[...], 0.0)
    out_ref[...] = (
        jnp.dot(h1, w2_ref[...], preferred_element_type=jnp.float32) * dis
    )


def _tc_c_body(p_ref, h_ref, dis_ref, b2_ref, out_ref):
    rows = pl.ds(0, _N)
    out_ref[...] = (
        dis_ref[rows, :]
        * (p_ref[0, rows, :] + p_ref[1, rows, :] - h_ref[rows, :])
        + b2_ref[...]
    )


_tc_a = pl.pallas_call(
    _tc_a_body,
    out_shape=[
        jax.ShapeDtypeStruct((_N_PAD, 1), jnp.float32),
        jax.ShapeDtypeStruct((_N_PAD, _D_HID), jnp.float32),
    ],
)

_tc_b = pl.pallas_call(
    _tc_b_body,
    out_shape=jax.ShapeDtypeStruct((_N_PAD, _D_OUT), jnp.float32),
)

_tc_c = pl.pallas_call(
    _tc_c_body,
    out_shape=jax.ShapeDtypeStruct((_N, _D_OUT), jnp.float32),
)


def kernel(x, edge_index, W1, b1, W2, b2):
    ei3 = edge_index.reshape(2, _NCHUNKS, _CHUNK)

    degp = _make_deg()(ei3)
    dis, h1s = _tc_a(degp, x, W1)
    p1 = _make_msg(_D_HID)(h1s, ei3)
    h2s = _tc_b(p1, h1s, dis, b1.reshape(1, _D_HID), W2)
    p2 = _make_msg(_D_OUT)(h2s, ei3)
    return _tc_c(p2, h2s, dis, b2.reshape(1, _D_OUT))


# speedup vs baseline: 2.0899x; 1.0079x over previous
import jax
import jax.numpy as jnp
from jax import lax
from jax.experimental import pallas as pl
from jax.experimental.pallas import tpu as pltpu
from jax.experimental.pallas import tpu_sc as plsc

_N = 10000
_E = 320000
_D_IN = 128
_D_HID = 32
_D_OUT = 16

_NC = 2
_NS = 16
_NW = _NC * _NS
_CHUNK = 128
_NB = 6
_NCHUNKS = _E // _CHUNK
_CPW = _NCHUNKS // _NW
_XTRA = _NCHUNKS - _CPW * _NW
_N_DEG = 10240
_DSTRIPE = _N_DEG // _NS
_STRIPE = _N // _NS
_R = 1000


def _wbase(wid):
    return wid * _CPW + jnp.minimum(wid, _XTRA)


def _load_idx(ei_hbm, which, wid, idx_v):
    base = _wbase(wid)
    pltpu.sync_copy(
        ei_hbm.at[which, pl.ds(base, _CPW)], idx_v.at[pl.ds(0, _CPW)]
    )

    @pl.when(wid < _XTRA)
    def _():
        pltpu.sync_copy(
            ei_hbm.at[which, pl.ds(base + _CPW, 1)], idx_v.at[pl.ds(_CPW, 1)]
        )


def _deg_body(ei_hbm, deg_hbm, dst_v, ones_v, buf_v, deg_sh, ssem):
    c = lax.axis_index("c")
    s = lax.axis_index("s")
    wid = c * _NS + s

    def fill_ones(i, _):
        ones_v[pl.ds(i * 16, 16)] = jnp.full((16,), 1.0, jnp.float32)
        return 0

    lax.fori_loop(0, _CHUNK // 16, fill_ones, 0)

    def fill_zero(i, _):
        buf_v[pl.ds(i * 16, 16)] = jnp.zeros((16,), jnp.float32)
        return 0

    lax.fori_loop(0, _DSTRIPE // 16, fill_zero, 0)

    stripe = pl.ds(s * _DSTRIPE, _DSTRIPE)
    pltpu.sync_copy(buf_v, deg_sh.at[stripe])
    _load_idx(ei_hbm, 1, wid, dst_v)
    plsc.subcore_barrier()

    def body(g, _):
        sds = []
        for b in range(_NB):
            j = g * _NB + b
            sds.append(
                pltpu.async_copy(ones_v, deg_sh.at[dst_v.at[j]], ssem, add=True)
            )
        for d_ in sds:
            d_.wait()
        return 0

    lax.fori_loop(0, _CPW // _NB, body, 0)

    @pl.when(wid < _XTRA)
    def _():
        pltpu.sync_copy(ones_v, deg_sh.at[dst_v.at[_CPW]], add=True)

    plsc.subcore_barrier()
    pltpu.sync_copy(deg_sh.at[stripe], buf_v)
    pltpu.sync_copy(buf_v, deg_hbm.at[c, stripe])


def _make_deg():
    mesh = plsc.VectorSubcoreMesh(core_axis_name="c", subcore_axis_name="s")
    return pl.kernel(
        _deg_body,
        out_type=jax.ShapeDtypeStruct((_NC, _N_DEG), jnp.float32),
        mesh=mesh,
        compiler_params=pltpu.CompilerParams(use_tc_tiling_on_sc=False),
        scratch_types=[
            pltpu.VMEM((_CPW + 1, _CHUNK), jnp.int32),
            pltpu.VMEM((_CHUNK,), jnp.float32),
            pltpu.VMEM((_DSTRIPE,), jnp.float32),
            pltpu.VMEM_SHARED((_N_DEG,), jnp.float32),
            pltpu.SemaphoreType.DMA,
        ],
    )


def _msg_body(h_hbm, ei_hbm, out_hbm, src_v, dst_v, rows_v, stage_v, acc_sh,
              gsem, ssem):
    c = lax.axis_index("c")
    s = lax.axis_index("s")
    wid = c * _NS + s
    stripe = pl.ds(s * _STRIPE, _STRIPE)

    pltpu.sync_copy(h_hbm.at[stripe], stage_v)
    pltpu.sync_copy(stage_v, acc_sh.at[stripe])
    _load_idx(ei_hbm, 0, wid, src_v)
    _load_idx(ei_hbm, 1, wid, dst_v)
    plsc.subcore_barrier()

    def body(g, _):
        gds = []
        for b in range(_NB):
            j = g * _NB + b
            gds.append(
                pltpu.async_copy(h_hbm.at[src_v.at[j]], rows_v.at[b], gsem)
            )
        sds = []
        for b in range(_NB):
            j = g * _NB + b
            gds[b].wait()
            sds.append(
                pltpu.async_copy(
                    rows_v.at[b], acc_sh.at[dst_v.at[j]], ssem, add=True
                )
            )
        for d_ in sds:
            d_.wait()
        return 0

    lax.fori_loop(0, _CPW // _NB, body, 0)

    @pl.when(wid < _XTRA)
    def _():
        pltpu.sync_copy(h_hbm.at[src_v.at[_CPW]], rows_v.at[0])
        pltpu.sync_copy(rows_v.at[0], acc_sh.at[dst_v.at[_CPW]], add=True)

    plsc.subcore_barrier()
    pltpu.sync_copy(acc_sh.at[stripe], stage_v)
    pltpu.sync_copy(stage_v, out_hbm.at[c, stripe])


def _make_msg(d):
    mesh = plsc.VectorSubcoreMesh(core_axis_name="c", subcore_axis_name="s")
    return pl.kernel(
        _msg_body,
        out_type=jax.ShapeDtypeStruct((_NC, _N, d), jnp.float32),
        mesh=mesh,
        compiler_params=pltpu.CompilerParams(use_tc_tiling_on_sc=False),
        scratch_types=[
            pltpu.VMEM((_CPW + 1, _CHUNK), jnp.int32),
            pltpu.VMEM((_CPW + 1, _CHUNK), jnp.int32),
            pltpu.VMEM((_NB, _CHUNK, d), jnp.float32),
            pltpu.VMEM((_STRIPE, d), jnp.float32),
            pltpu.VMEM_SHARED((_N, d), jnp.float32),
            pltpu.SemaphoreType.DMA,
            pltpu.SemaphoreType.DMA,
        ],
    )


def _tc_a_body(degt_ref, x_ref, w1_ref, dis_ref, h_ref):
    deg = degt_ref[:, 0:1] + degt_ref[:, 1:2] + 1.0
    dis = lax.rsqrt(deg)
    h = jnp.dot(x_ref[...], w1_ref[...], preferred_element_type=jnp.float32)
    dis_ref[...] = dis
    h_ref[...] = h * dis


def _tc_b_body(p_ref, h_ref, dis_ref, b1_ref, w2_ref, out_ref):
    dis = dis_ref[...]
    acc = p_ref[0, :, :] + p_ref[1, :, :] - h_ref[...]
    h1 = jnp.maximum(dis * acc + b1_ref[...], 0.0)
    out_ref[...] = (
        jnp.dot(h1, w2_ref[...], preferred_element_type=jnp.float32) * dis
    )


def _tc_c_body(p_ref, h_ref, dis_ref, b2_ref, out_ref):
    out_ref[...] = (
        dis_ref[...] * (p_ref[0, :, :] + p_ref[1, :, :] - h_ref[...])
        + b2_ref[...]
    )


_GRID = _N // _R

_tc_a = pl.pallas_call(
    _tc_a_body,
    grid=(_GRID,),
    in_specs=[
        pl.BlockSpec((_R, _NC), lambda i: (i, 0)),
        pl.BlockSpec((_R, _D_IN), lambda i: (i, 0)),
        pl.BlockSpec((_D_IN, _D_HID), lambda i: (0, 0)),
    ],
    out_specs=[
        pl.BlockSpec((_R, 1), lambda i: (i, 0)),
        pl.BlockSpec((_R, _D_HID), lambda i: (i, 0)),
    ],
    out_shape=[
        jax.ShapeDtypeStruct((_N, 1), jnp.float32),
        jax.ShapeDtypeStruct((_N, _D_HID), jnp.float32),
    ],
)

_tc_b = pl.pallas_call(
    _tc_b_body,
    grid=(_GRID,),
    in_specs=[
        pl.BlockSpec((_NC, _R, _D_HID), lambda i: (0, i, 0)),
        pl.BlockSpec((_R, _D_HID), lambda i: (i, 0)),
        pl.BlockSpec((_R, 1), lambda i: (i, 0)),
        pl.BlockSpec((1, _D_HID), lambda i: (0, 0)),
        pl.BlockSpec((_D_HID, _D_OUT), lambda i: (0, 0)),
    ],
    out_specs=pl.BlockSpec((_R, _D_OUT), lambda i: (i, 0)),
    out_shape=jax.ShapeDtypeStruct((_N, _D_OUT), jnp.float32),
)

_tc_c = pl.pallas_call(
    _tc_c_body,
    grid=(_GRID,),
    in_specs=[
        pl.BlockSpec((_NC, _R, _D_OUT), lambda i: (0, i, 0)),
        pl.BlockSpec((_R, _D_OUT), lambda i: (i, 0)),
        pl.BlockSpec((_R, 1), lambda i: (i, 0)),
        pl.BlockSpec((1, _D_OUT), lambda i: (0, 0)),
    ],
    out_specs=pl.BlockSpec((_R, _D_OUT), lambda i: (i, 0)),
    out_shape=jax.ShapeDtypeStruct((_N, _D_OUT), jnp.float32),
)


def kernel(x, edge_index, W1, b1, W2, b2):
    ei3 = edge_index.reshape(2, _NCHUNKS, _CHUNK)

    degp = _make_deg()(ei3)
    dis, h1s = _tc_a(degp[:, :_N].T, x, W1)
    p1 = _make_msg(_D_HID)(h1s, ei3)
    h2s = _tc_b(p1, h1s, dis, b1.reshape(1, _D_HID), W2)
    p2 = _make_msg(_D_OUT)(h2s, ei3)
    return _tc_c(p2, h2s, dis, b2.reshape(1, _D_OUT))


# speedup vs baseline: 2.2990x; 1.1000x over previous
import jax
import jax.numpy as jnp
from jax import lax
from jax.experimental import pallas as pl
from jax.experimental.pallas import tpu as pltpu
from jax.experimental.pallas import tpu_sc as plsc

_N = 10000
_E = 320000
_D_IN = 128
_D_HID = 32
_D_OUT = 16

_NC = 2
_NS = 16
_NW = _NC * _NS
_CHUNK = 128
_NB = 13
_NCHUNKS = _E // _CHUNK
_CPW = _NCHUNKS // _NW
_XTRA = _NCHUNKS - _CPW * _NW
_N_DEG = 10240
_DSTRIPE = _N_DEG // _NS
_STRIPE = _N // _NS
_R = 1000


def _wbase(wid):
    return wid * _CPW + jnp.minimum(wid, _XTRA)


def _load_idx(ei_hbm, which, wid, idx_v):
    base = _wbase(wid)
    pltpu.sync_copy(
        ei_hbm.at[which, pl.ds(base, _CPW)], idx_v.at[pl.ds(0, _CPW)]
    )

    @pl.when(wid < _XTRA)
    def _():
        pltpu.sync_copy(
            ei_hbm.at[which, pl.ds(base + _CPW, 1)], idx_v.at[pl.ds(_CPW, 1)]
        )


def _deg_body(ei_hbm, deg_hbm, dst_v, ones_v, buf_v, deg_sh, ssem):
    c = lax.axis_index("c")
    s = lax.axis_index("s")
    wid = c * _NS + s

    def fill_ones(i, _):
        ones_v[pl.ds(i * 16, 16)] = jnp.full((16,), 1.0, jnp.float32)
        return 0

    lax.fori_loop(0, _CHUNK // 16, fill_ones, 0)

    def fill_zero(i, _):
        buf_v[pl.ds(i * 16, 16)] = jnp.zeros((16,), jnp.float32)
        return 0

    lax.fori_loop(0, _DSTRIPE // 16, fill_zero, 0)

    stripe = pl.ds(s * _DSTRIPE, _DSTRIPE)
    pltpu.sync_copy(buf_v, deg_sh.at[stripe])
    _load_idx(ei_hbm, 1, wid, dst_v)
    plsc.subcore_barrier()

    def body(g, _):
        sds = []
        for b in range(_NB):
            j = g * _NB + b
            sds.append(
                pltpu.async_copy(ones_v, deg_sh.at[dst_v.at[j]], ssem, add=True)
            )
        for d_ in sds:
            d_.wait()
        return 0

    lax.fori_loop(0, _CPW // _NB, body, 0)

    @pl.when(wid < _XTRA)
    def _():
        pltpu.sync_copy(ones_v, deg_sh.at[dst_v.at[_CPW]], add=True)

    plsc.subcore_barrier()
    pltpu.sync_copy(deg_sh.at[stripe], buf_v)
    pltpu.sync_copy(buf_v, deg_hbm.at[c, stripe])


def _make_deg():
    mesh = plsc.VectorSubcoreMesh(core_axis_name="c", subcore_axis_name="s")
    return pl.kernel(
        _deg_body,
        out_type=jax.ShapeDtypeStruct((_NC, _N_DEG), jnp.float32),
        mesh=mesh,
        compiler_params=pltpu.CompilerParams(use_tc_tiling_on_sc=False),
        scratch_types=[
            pltpu.VMEM((_CPW + 1, _CHUNK), jnp.int32),
            pltpu.VMEM((_CHUNK,), jnp.float32),
            pltpu.VMEM((_DSTRIPE,), jnp.float32),
            pltpu.VMEM_SHARED((_N_DEG,), jnp.float32),
            pltpu.SemaphoreType.DMA,
        ],
    )


def _msg_body(h_hbm, ei_hbm, out_hbm, src_v, dst_v, rows_v, stage_v, acc_sh,
              gsem, ssem):
    c = lax.axis_index("c")
    s = lax.axis_index("s")
    wid = c * _NS + s
    stripe = pl.ds(s * _STRIPE, _STRIPE)

    pltpu.sync_copy(h_hbm.at[stripe], stage_v)
    pltpu.sync_copy(stage_v, acc_sh.at[stripe])
    _load_idx(ei_hbm, 0, wid, src_v)
    _load_idx(ei_hbm, 1, wid, dst_v)
    plsc.subcore_barrier()

    def body(g, _):
        gds = []
        for b in range(_NB):
            j = g * _NB + b
            gds.append(
                pltpu.async_copy(h_hbm.at[src_v.at[j]], rows_v.at[b], gsem)
            )
        sds = []
        for b in range(_NB):
            j = g * _NB + b
            gds[b].wait()
            sds.append(
                pltpu.async_copy(
                    rows_v.at[b], acc_sh.at[dst_v.at[j]], ssem, add=True
                )
            )
        for d_ in sds:
            d_.wait()
        return 0

    lax.fori_loop(0, _CPW // _NB, body, 0)

    @pl.when(wid < _XTRA)
    def _():
        pltpu.sync_copy(h_hbm.at[src_v.at[_CPW]], rows_v.at[0])
        pltpu.sync_copy(rows_v.at[0], acc_sh.at[dst_v.at[_CPW]], add=True)

    plsc.subcore_barrier()
    pltpu.sync_copy(acc_sh.at[stripe], stage_v)
    pltpu.sync_copy(stage_v, out_hbm.at[c, stripe])


def _make_msg(d):
    mesh = plsc.VectorSubcoreMesh(core_axis_name="c", subcore_axis_name="s")
    return pl.kernel(
        _msg_body,
        out_type=jax.ShapeDtypeStruct((_NC, _N, d), jnp.float32),
        mesh=mesh,
        compiler_params=pltpu.CompilerParams(use_tc_tiling_on_sc=False),
        scratch_types=[
            pltpu.VMEM((_CPW + 1, _CHUNK), jnp.int32),
            pltpu.VMEM((_CPW + 1, _CHUNK), jnp.int32),
            pltpu.VMEM((_NB, _CHUNK, d), jnp.float32),
            pltpu.VMEM((_STRIPE, d), jnp.float32),
            pltpu.VMEM_SHARED((_N, d), jnp.float32),
            pltpu.SemaphoreType.DMA,
            pltpu.SemaphoreType.DMA,
        ],
    )


def _tc_a_body(degt_ref, x_ref, w1_ref, dis_ref, h_ref):
    deg = degt_ref[:, 0:1] + degt_ref[:, 1:2] + 1.0
    dis = lax.rsqrt(deg)
    h = jnp.dot(x_ref[...], w1_ref[...], preferred_element_type=jnp.float32)
    dis_ref[...] = dis
    h_ref[...] = h * dis


_tc_a = pl.pallas_call(
    _tc_a_body,
    out_shape=[
        jax.ShapeDtypeStruct((_N, 1), jnp.float32),
        jax.ShapeDtypeStruct((_N, _D_HID), jnp.float32),
    ],
)


def _tc_b_body(p_ref, h_ref, dis_ref, b1_ref, w2_ref, out_ref):
    dis = dis_ref[...]
    acc = p_ref[0, :, :] + p_ref[1, :, :] - h_ref[...]
    h1 = jnp.maximum(dis * acc + b1_ref[...], 0.0)
    out_ref[...] = (
        jnp.dot(h1, w2_ref[...], preferred_element_type=jnp.float32) * dis
    )


def _tc_c_body(p_ref, h_ref, dis_ref, b2_ref, out_ref):
    out_ref[...] = (
        dis_ref[...] * (p_ref[0, :, :] + p_ref[1, :, :] - h_ref[...])
        + b2_ref[...]
    )


_tc_b = pl.pallas_call(
    _tc_b_body,
    out_shape=jax.ShapeDtypeStruct((_N, _D_OUT), jnp.float32),
)

_tc_c = pl.pallas_call(
    _tc_c_body,
    out_shape=jax.ShapeDtypeStruct((_N, _D_OUT), jnp.float32),
)


def kernel(x, edge_index, W1, b1, W2, b2):
    ei3 = edge_index.reshape(2, _NCHUNKS, _CHUNK)

    degp = _make_deg()(ei3)
    dis, h1s = _tc_a(degp[:, :_N].T, x, W1)
    p1 = _make_msg(_D_HID)(h1s, ei3)
    h2s = _tc_b(p1, h1s, dis, b1.reshape(1, _D_HID), W2)
    p2 = _make_msg(_D_OUT)(h2s, ei3)
    return _tc_c(p2, h2s, dis, b2.reshape(1, _D_OUT))


# speedup vs baseline: 2.3379x; 1.0169x over previous
import functools

import jax
import jax.numpy as jnp
from jax import lax
from jax.experimental import pallas as pl
from jax.experimental.pallas import tpu as pltpu
from jax.experimental.pallas import tpu_sc as plsc

_N = 10000
_E = 320000
_D_IN = 128
_D_HID = 32
_D_OUT = 16

_NC = 2
_NS = 16
_NW = _NC * _NS
_CHUNK = 128
_NB = 13


def _nb(d):
    return 26 if d <= 16 else 13
_NCHUNKS = _E // _CHUNK
_CPW = _NCHUNKS // _NW
_XTRA = _NCHUNKS - _CPW * _NW
_N_DEG = 10240
_DSTRIPE = _N_DEG // _NS
_STRIPE = _N // _NS
_R = 1000


def _wbase(wid):
    return wid * _CPW + jnp.minimum(wid, _XTRA)


def _load_idx(ei_hbm, which, wid, idx_v):
    base = _wbase(wid)
    pltpu.sync_copy(
        ei_hbm.at[which, pl.ds(base, _CPW)], idx_v.at[pl.ds(0, _CPW)]
    )

    @pl.when(wid < _XTRA)
    def _():
        pltpu.sync_copy(
            ei_hbm.at[which, pl.ds(base + _CPW, 1)], idx_v.at[pl.ds(_CPW, 1)]
        )


def _deg_body(ei_hbm, deg_hbm, dst_v, ones_v, buf_v, deg_sh, ssem):
    c = lax.axis_index("c")
    s = lax.axis_index("s")
    wid = c * _NS + s

    def fill_ones(i, _):
        ones_v[pl.ds(i * 16, 16)] = jnp.full((16,), 1.0, jnp.float32)
        return 0

    lax.fori_loop(0, _CHUNK // 16, fill_ones, 0)

    def fill_zero(i, _):
        buf_v[pl.ds(i * 16, 16)] = jnp.zeros((16,), jnp.float32)
        return 0

    lax.fori_loop(0, _DSTRIPE // 16, fill_zero, 0)

    stripe = pl.ds(s * _DSTRIPE, _DSTRIPE)
    pltpu.sync_copy(buf_v, deg_sh.at[stripe])
    _load_idx(ei_hbm, 1, wid, dst_v)
    plsc.subcore_barrier()

    def body(g, _):
        sds = []
        for b in range(_NB):
            j = g * _NB + b
            sds.append(
                pltpu.async_copy(ones_v, deg_sh.at[dst_v.at[j]], ssem, add=True)
            )
        for d_ in sds:
            d_.wait()
        return 0

    lax.fori_loop(0, _CPW // _NB, body, 0)

    @pl.when(wid < _XTRA)
    def _():
        pltpu.sync_copy(ones_v, deg_sh.at[dst_v.at[_CPW]], add=True)

    plsc.subcore_barrier()
    pltpu.sync_copy(deg_sh.at[stripe], buf_v)
    pltpu.sync_copy(buf_v, deg_hbm.at[c, stripe])


def _make_deg():
    mesh = plsc.VectorSubcoreMesh(core_axis_name="c", subcore_axis_name="s")
    return pl.kernel(
        _deg_body,
        out_type=jax.ShapeDtypeStruct((_NC, _N_DEG), jnp.float32),
        mesh=mesh,
        compiler_params=pltpu.CompilerParams(use_tc_tiling_on_sc=False),
        scratch_types=[
            pltpu.VMEM((_CPW + 1, _CHUNK), jnp.int32),
            pltpu.VMEM((_CHUNK,), jnp.float32),
            pltpu.VMEM((_DSTRIPE,), jnp.float32),
            pltpu.VMEM_SHARED((_N_DEG,), jnp.float32),
            pltpu.SemaphoreType.DMA,
        ],
    )


def _msg_body(nb, h_hbm, ei_hbm, out_hbm, src_v, dst_v, rows_v, stage_v,
              acc_sh, gsem, ssem):
    c = lax.axis_index("c")
    s = lax.axis_index("s")
    wid = c * _NS + s
    stripe = pl.ds(s * _STRIPE, _STRIPE)

    pltpu.sync_copy(h_hbm.at[stripe], stage_v)
    pltpu.sync_copy(stage_v, acc_sh.at[stripe])
    _load_idx(ei_hbm, 0, wid, src_v)
    _load_idx(ei_hbm, 1, wid, dst_v)
    plsc.subcore_barrier()

    def body(g, _):
        gds = []
        for b in range(nb):
            j = g * nb + b
            gds.append(
                pltpu.async_copy(h_hbm.at[src_v.at[j]], rows_v.at[b], gsem)
            )
        sds = []
        for b in range(nb):
            j = g * nb + b
            gds[b].wait()
            sds.append(
                pltpu.async_copy(
                    rows_v.at[b], acc_sh.at[dst_v.at[j]], ssem, add=True
                )
            )
        for d_ in sds:
            d_.wait()
        return 0

    lax.fori_loop(0, _CPW // nb, body, 0)

    @pl.when(wid < _XTRA)
    def _():
        pltpu.sync_copy(h_hbm.at[src_v.at[_CPW]], rows_v.at[0])
        pltpu.sync_copy(rows_v.at[0], acc_sh.at[dst_v.at[_CPW]], add=True)

    plsc.subcore_barrier()
    pltpu.sync_copy(acc_sh.at[stripe], stage_v)
    pltpu.sync_copy(stage_v, out_hbm.at[c, stripe])


def _make_msg(d):
    mesh = plsc.VectorSubcoreMesh(core_axis_name="c", subcore_axis_name="s")
    return pl.kernel(
        functools.partial(_msg_body, _nb(d)),
        out_type=jax.ShapeDtypeStruct((_NC, _N, d), jnp.float32),
        mesh=mesh,
        compiler_params=pltpu.CompilerParams(use_tc_tiling_on_sc=False),
        scratch_types=[
            pltpu.VMEM((_CPW + 1, _CHUNK), jnp.int32),
            pltpu.VMEM((_CPW + 1, _CHUNK), jnp.int32),
            pltpu.VMEM((_nb(d), _CHUNK, d), jnp.float32),
            pltpu.VMEM((_STRIPE, d), jnp.float32),
            pltpu.VMEM_SHARED((_N, d), jnp.float32),
            pltpu.SemaphoreType.DMA,
            pltpu.SemaphoreType.DMA,
        ],
    )


def _tc_a_body(degt_ref, x_ref, w1_ref, dis_ref, h_ref):
    deg = degt_ref[:, 0:1] + degt_ref[:, 1:2] + 1.0
    dis = lax.rsqrt(deg)
    h = jnp.dot(x_ref[...], w1_ref[...], preferred_element_type=jnp.float32)
    dis_ref[...] = dis
    h_ref[...] = h * dis


_tc_a = pl.pallas_call(
    _tc_a_body,
    out_shape=[
        jax.ShapeDtypeStruct((_N, 1), jnp.float32),
        jax.ShapeDtypeStruct((_N, _D_HID), jnp.float32),
    ],
)


def _tc_b_body(p_ref, h_ref, dis_ref, b1_ref, w2_ref, out_ref):
    dis = dis_ref[...]
    acc = p_ref[0, :, :] + p_ref[1, :, :] - h_ref[...]
    h1 = jnp.maximum(dis * acc + b1_ref[...], 0.0)
    out_ref[...] = (
        jnp.dot(h1, w2_ref[...], preferred_element_type=jnp.float32) * dis
    )


def _tc_c_body(p_ref, h_ref, dis_ref, b2_ref, out_ref):
    out_ref[...] = (
        dis_ref[...] * (p_ref[0, :, :] + p_ref[1, :, :] - h_ref[...])
        + b2_ref[...]
    )


_tc_b = pl.pallas_call(
    _tc_b_body,
    out_shape=jax.ShapeDtypeStruct((_N, _D_OUT), jnp.float32),
)

_tc_c = pl.pallas_call(
    _tc_c_body,
    out_shape=jax.ShapeDtypeStruct((_N, _D_OUT), jnp.float32),
)


def kernel(x, edge_index, W1, b1, W2, b2):
    ei3 = edge_index.reshape(2, _NCHUNKS, _CHUNK)

    degp = _make_deg()(ei3)
    dis, h1s = _tc_a(degp[:, :_N].T, x, W1)
    p1 = _make_msg(_D_HID)(h1s, ei3)
    h2s = _tc_b(p1, h1s, dis, b1.reshape(1, _D_HID), W2)
    p2 = _make_msg(_D_OUT)(h2s, ei3)
    return _tc_c(p2, h2s, dis, b2.reshape(1, _D_OUT))
